# baseline (speedup 1.0000x reference)
import functools

import jax
import jax.numpy as jnp
from jax import lax
from jax.experimental import pallas as pl
from jax.experimental.pallas import tpu as pltpu, tpu_sc as plsc

_NC = 2
_NS = 16
_L = 16
_NW = _NC * _NS


def _lse_body(x_ref, lse_ref):
    x = x_ref[...]
    m = jnp.max(x, axis=1, keepdims=True)
    s = jnp.sum(jnp.exp(x - m), axis=1, keepdims=True)
    lse_ref[...] = m + jnp.log(s)


def _make_sc_gather(V, Vp, B, Tp):
    BPW = B // _NW
    mesh = plsc.VectorSubcoreMesh(core_axis_name="c", subcore_axis_name="s")

    @functools.partial(
        pl.kernel,
        out_type=jax.ShapeDtypeStruct((B, Tp, Vp), jnp.float32),
        mesh=mesh,
        compiler_params=pltpu.CompilerParams(needs_layout_passes=False),
        scratch_types=[
            pltpu.VMEM((BPW, Tp), jnp.int32),
            pltpu.VMEM((Tp, Vp), jnp.float32),
            pltpu.SemaphoreType.DMA,
        ],
    )
    def sc_gather(tpad_hbm, in2d_hbm, out_hbm, idx2d_v, rows_v, gsem):
        wid = lax.axis_index("s") * _NC + lax.axis_index("c")
        b0 = wid * BPW
        pltpu.sync_copy(in2d_hbm.at[pl.ds(b0, BPW)], idx2d_v)

        def chunk_step(g, carry):
            pltpu.async_copy(tpad_hbm.at[idx2d_v.at[g]], rows_v, gsem).wait()
            pltpu.sync_copy(rows_v, out_hbm.at[b0 + g])
            return carry

        lax.fori_loop(0, BPW, chunk_step, 0)

    return sc_gather


def _make_sc_loss(V, BT):
    PW = BT // _NW
    NIDX = 128
    NR = (PW + NIDX - 1) // NIDX
    PWP = NR * NIDX
    mesh = plsc.VectorSubcoreMesh(core_axis_name="c", subcore_axis_name="s")

    @functools.partial(
        pl.kernel,
        out_type=jax.ShapeDtypeStruct((_NW, _L), jnp.float32),
        mesh=mesh,
        compiler_params=pltpu.CompilerParams(
            needs_layout_passes=False, use_tc_tiling_on_sc=False
        ),
        scratch_types=[
            pltpu.VMEM((PW,), jnp.int32),
            pltpu.VMEM((PW,), jnp.int32),
            pltpu.VMEM((1024,), jnp.float32),
            pltpu.VMEM((NR, NIDX), jnp.int32),
            pltpu.VMEM((NR, NIDX, 1), jnp.float32),
            pltpu.VMEM((_L,), jnp.float32),
            pltpu.SemaphoreType.DMA,
        ],
    )
    def sc_loss(tflat_hbm, inflat_hbm, tgt_hbm, lse_hbm, part_hbm,
                idx_v, tgt_v, lse_v, eidx_v, picks_v, part_v, gsem):
        wid = lax.axis_index("s") * _NC + lax.axis_index("c")
        pltpu.sync_copy(inflat_hbm.at[pl.ds(wid * PW, PW)], idx_v)
        pltpu.sync_copy(tgt_hbm.at[pl.ds(wid * PW, PW)], tgt_v)
        pltpu.sync_copy(lse_hbm, lse_v)

        def eidx_step(s, carry):
            p = s * _L + lax.iota(jnp.int32, _L)
            pc = jnp.minimum(p, PW - 1)
            iv = plsc.load_gather(idx_v, [pc])
            tv = plsc.load_gather(tgt_v, [pc])
            ev = jnp.where(p < PW, iv * V + tv, 0)
            plsc.store_scatter(eidx_v, [p >> 7, p & (NIDX - 1)], ev)
            return carry

        lax.fori_loop(0, PWP // _L, eidx_step, 0)

        for j in range(NR):
            pltpu.async_copy(tflat_hbm.at[eidx_v.at[j]], picks_v.at[j], gsem)
        for j in range(NR):
            pltpu.make_async_copy(tflat_hbm.at[eidx_v.at[j]], picks_v.at[j], gsem).wait()

        def acc_step(s, acc):
            p = s * _L + lax.iota(jnp.int32, _L)
            iv = plsc.load_gather(idx_v, [p])
            acc = acc + plsc.load_gather(lse_v, [iv])
            pk = plsc.load_gather(picks_v, [p >> 7, p & (NIDX - 1),
                                            jnp.zeros((_L,), jnp.int32)])
            return acc - pk

        acc = lax.fori_loop(0, PW // _L, acc_step, jnp.zeros((_L,), jnp.float32))

        part_v[...] = acc
        pltpu.sync_copy(part_v, part_hbm.at[wid])

    return sc_loss


def kernel(table, inputs, targets):
    V = table.shape[0]
    Vp = 1024
    B, T = inputs.shape
    BT = B * T
    Tp = (T + 7) // 8 * 8
    in2d = inputs.astype(jnp.int32)
    in2d_pad = jnp.pad(in2d, ((0, 0), (0, Tp - T)))
    inflat = in2d.reshape(BT)
    tgt = targets.reshape(BT).astype(jnp.int32)
    tpad = jnp.pad(table, ((0, 0), (0, Vp - V)))
    tflat = table.reshape(V * V, 1)

    lse = pl.pallas_call(
        _lse_body,
        out_shape=jax.ShapeDtypeStruct((V, 1), jnp.float32),
    )(table)
    lse_pad = jnp.pad(lse.reshape(V), (0, 1024 - V))

    logits_pad = _make_sc_gather(V, Vp, B, Tp)(tpad, in2d_pad)
    logits = lax.slice(logits_pad, (0, 0, 0), (B, T, V))
    parts = _make_sc_loss(V, BT)(tflat, inflat, tgt, lse_pad)
    loss = jnp.sum(parts / BT)
    return logits, loss


# speedup vs baseline: 1.1009x; 1.1009x over previous
import functools

import jax
import jax.numpy as jnp
from jax import lax
from jax.experimental import pallas as pl
from jax.experimental.pallas import tpu as pltpu, tpu_sc as plsc

_NC = 2
_NS = 16
_L = 16
_NW = _NC * _NS


def _lse_body(x_ref, lse_ref):
    x = x_ref[...]
    m = jnp.max(x, axis=1, keepdims=True)
    s = jnp.sum(jnp.exp(x - m), axis=1, keepdims=True)
    lse_ref[...] = m + jnp.log(s)


def _make_sc_gather(Vp, BT, CH):
    PW = BT // _NW
    NCH = PW // CH
    mesh = plsc.VectorSubcoreMesh(core_axis_name="c", subcore_axis_name="s")

    @functools.partial(
        pl.kernel,
        out_type=jax.ShapeDtypeStruct((BT, Vp), jnp.float32),
        mesh=mesh,
        compiler_params=pltpu.CompilerParams(
            needs_layout_passes=False, use_tc_tiling_on_sc=False
        ),
        scratch_types=[
            pltpu.VMEM((PW,), jnp.int32),
            pltpu.VMEM((CH, Vp), jnp.float32),
            pltpu.SemaphoreType.DMA,
        ],
    )
    def sc_gather(tpad_hbm, inflat_hbm, out_hbm, idx_v, rows_v, gsem):
        wid = lax.axis_index("s") * _NC + lax.axis_index("c")
        base = wid * PW
        pltpu.sync_copy(inflat_hbm.at[pl.ds(base, PW)], idx_v)

        def chunk_step(g, carry):
            pltpu.async_copy(
                tpad_hbm.at[idx_v.at[pl.ds(g * CH, CH)]], rows_v, gsem
            ).wait()
            pltpu.sync_copy(rows_v, out_hbm.at[pl.ds(base + g * CH, CH)])
            return carry

        lax.fori_loop(0, NCH, chunk_step, 0)

    return sc_gather


def _make_relayout(V, Vp, B, T, NB):
    def body(x_ref, o_ref):
        x = x_ref[...].reshape(NB * T, Vp)
        o_ref[...] = x[:, :V].reshape(NB, T, V)

    return pl.pallas_call(
        body,
        grid=(B // NB,),
        in_specs=[pl.BlockSpec((NB * T * Vp,), lambda i: (i,))],
        out_specs=pl.BlockSpec((NB, T, V), lambda i: (i, 0, 0)),
        out_shape=jax.ShapeDtypeStruct((B, T, V), jnp.float32),
        compiler_params=pltpu.CompilerParams(
            dimension_semantics=("arbitrary",),
        ),
    )


def _make_sc_loss(V, BT):
    PW = BT // _NW
    NIDX = 128
    NR = (PW + NIDX - 1) // NIDX
    PWP = NR * NIDX
    mesh = plsc.VectorSubcoreMesh(core_axis_name="c", subcore_axis_name="s")

    @functools.partial(
        pl.kernel,
        out_type=jax.ShapeDtypeStruct((_NW, _L), jnp.float32),
        mesh=mesh,
        compiler_params=pltpu.CompilerParams(
            needs_layout_passes=False, use_tc_tiling_on_sc=False
        ),
        scratch_types=[
            pltpu.VMEM((PW,), jnp.int32),
            pltpu.VMEM((PW,), jnp.int32),
            pltpu.VMEM((1024,), jnp.float32),
            pltpu.VMEM((NR, NIDX), jnp.int32),
            pltpu.VMEM((NR, NIDX, 1), jnp.float32),
            pltpu.VMEM((_L,), jnp.float32),
            pltpu.SemaphoreType.DMA,
        ],
    )
    def sc_loss(tflat_hbm, inflat_hbm, tgt_hbm, lse_hbm, part_hbm,
                idx_v, tgt_v, lse_v, eidx_v, picks_v, part_v, gsem):
        wid = lax.axis_index("s") * _NC + lax.axis_index("c")
        pltpu.sync_copy(inflat_hbm.at[pl.ds(wid * PW, PW)], idx_v)
        pltpu.sync_copy(tgt_hbm.at[pl.ds(wid * PW, PW)], tgt_v)
        pltpu.sync_copy(lse_hbm, lse_v)

        def eidx_step(s, carry):
            p = s * _L + lax.iota(jnp.int32, _L)
            pc = jnp.minimum(p, PW - 1)
            iv = plsc.load_gather(idx_v, [pc])
            tv = plsc.load_gather(tgt_v, [pc])
            ev = jnp.where(p < PW, iv * V + tv, 0)
            plsc.store_scatter(eidx_v, [p >> 7, p & (NIDX - 1)], ev)
            return carry

        lax.fori_loop(0, PWP // _L, eidx_step, 0)

        for j in range(NR):
            pltpu.async_copy(tflat_hbm.at[eidx_v.at[j]], picks_v.at[j], gsem)
        for j in range(NR):
            pltpu.make_async_copy(tflat_hbm.at[eidx_v.at[j]], picks_v.at[j], gsem).wait()

        def acc_step(s, acc):
            p = s * _L + lax.iota(jnp.int32, _L)
            iv = plsc.load_gather(idx_v, [p])
            acc = acc + plsc.load_gather(lse_v, [iv])
            pk = plsc.load_gather(picks_v, [p >> 7, p & (NIDX - 1),
                                            jnp.zeros((_L,), jnp.int32)])
            return acc - pk

        acc = lax.fori_loop(0, PW // _L, acc_step, jnp.zeros((_L,), jnp.float32))

        part_v[...] = acc
        pltpu.sync_copy(part_v, part_hbm.at[wid])

    return sc_loss


def kernel(table, inputs, targets):
    V = table.shape[0]
    Vp = 1024
    B, T = inputs.shape
    BT = B * T
    inflat = inputs.astype(jnp.int32).reshape(BT)
    tgt = targets.reshape(BT).astype(jnp.int32)
    tpad = jnp.pad(table, ((0, 0), (0, Vp - V)))
    tflat = table.reshape(V * V, 1)

    lse = pl.pallas_call(
        _lse_body,
        out_shape=jax.ShapeDtypeStruct((V, 1), jnp.float32),
    )(table)
    lse_pad = jnp.pad(lse.reshape(V), (0, 1024 - V))

    staging = _make_sc_gather(Vp, BT, 64)(tpad, inflat)
    logits = _make_relayout(V, Vp, B, T, 16)(staging.reshape(BT * Vp))
    parts = _make_sc_loss(V, BT)(tflat, inflat, tgt, lse_pad)
    loss = jnp.sum(parts / BT)
    return logits, loss


# speedup vs baseline: 2.7685x; 2.5148x over previous
import functools

import jax
import jax.numpy as jnp
from jax import lax
from jax.experimental import pallas as pl
from jax.experimental.pallas import tpu as pltpu, tpu_sc as plsc

_NC = 2
_NS = 16
_L = 16
_NW = _NC * _NS


def _lse_body(x_ref, lse_ref):
    x = x_ref[...]
    m = jnp.max(x, axis=1, keepdims=True)
    s = jnp.sum(jnp.exp(x - m), axis=1, keepdims=True)
    lse_ref[...] = m + jnp.log(s)


def _make_sc_main(V, BT, CH):
    PW = BT // _NW
    NCH = PW // CH
    NRV = (PW + 127) // 128
    mesh = plsc.VectorSubcoreMesh(core_axis_name="c", subcore_axis_name="s")

    @functools.partial(
        pl.kernel,
        out_type=(
            jax.ShapeDtypeStruct((BT, 8, 128), jnp.float32),
            jax.ShapeDtypeStruct((_NW, _L), jnp.float32),
        ),
        mesh=mesh,
        compiler_params=pltpu.CompilerParams(needs_layout_passes=False),
        scratch_types=[
            pltpu.VMEM((NCH, CH), jnp.int32),
            pltpu.VMEM((NRV, 128), jnp.int32),
            pltpu.VMEM((NRV, 128), jnp.int32),
            pltpu.VMEM((1024,), jnp.float32),
            pltpu.VMEM((CH, 8, 128), jnp.float32),
            pltpu.VMEM((_L,), jnp.float32),
            pltpu.SemaphoreType.DMA,
        ],
    )
    def sc_main(tsw_hbm, ing_hbm, inv_hbm, tgt_hbm, lse_hbm,
                out_hbm, part_hbm,
                idxg_v, idx_v, tgt_v, lse_v, rows_v, part_v, gsem):
        wid = lax.axis_index("s") * _NC + lax.axis_index("c")
        base = wid * PW
        pltpu.sync_copy(ing_hbm.at[wid], idxg_v)
        pltpu.sync_copy(inv_hbm.at[wid], idx_v)
        pltpu.sync_copy(tgt_hbm.at[wid], tgt_v)
        pltpu.sync_copy(lse_hbm, lse_v)

        def lse_step(s, acc):
            p = s * _L + lax.iota(jnp.int32, _L)
            iv = plsc.load_gather(idx_v, [p >> 7, p & 127])
            return acc + plsc.load_gather(lse_v, [iv])

        acc = lax.fori_loop(0, PW // _L, lse_step, jnp.zeros((_L,), jnp.float32))

        def chunk_step(g, acc):
            pltpu.async_copy(tsw_hbm.at[idxg_v.at[g]], rows_v, gsem).wait()
            for h in range(CH // _L):
                jvec = h * _L + lax.iota(jnp.int32, _L)
                p = g * CH + jvec
                tv = plsc.load_gather(tgt_v, [p >> 7, p & 127])
                pick = plsc.load_gather(rows_v, [jvec, tv >> 7, tv & 127])
                acc = acc - pick
            pltpu.sync_copy(rows_v, out_hbm.at[pl.ds(base + g * CH, CH)])
            return acc

        acc = lax.fori_loop(0, NCH, chunk_step, acc)

        part_v[...] = acc
        pltpu.sync_copy(part_v, part_hbm.at[wid])

    return sc_main


def _make_relayout(V, B, T, NB):
    def body(x_ref, o_ref):
        x = x_ref[...].reshape(NB * T, 1024)
        o_ref[...] = x[:, :V].reshape(NB, T, V)

    return pl.pallas_call(
        body,
        grid=(B // NB,),
        in_specs=[pl.BlockSpec((NB * T, 8, 128), lambda i: (i, 0, 0))],
        out_specs=pl.BlockSpec((NB, T, V), lambda i: (i, 0, 0)),
        out_shape=jax.ShapeDtypeStruct((B, T, V), jnp.float32),
        compiler_params=pltpu.CompilerParams(
            dimension_semantics=("arbitrary",),
        ),
    )


def kernel(table, inputs, targets):
    V = table.shape[0]
    B, T = inputs.shape
    BT = B * T
    PW = BT // _NW
    NRV = (PW + 127) // 128
    tsw = jnp.pad(table, ((0, 0), (0, 1024 - V))).reshape(V, 8, 128)

    in_w = inputs.astype(jnp.int32).reshape(_NW, PW)
    tg_w = targets.astype(jnp.int32).reshape(_NW, PW)
    in_g = in_w.reshape(_NW, PW // 64, 64)
    pad = NRV * 128 - PW
    in_v = jnp.pad(in_w, ((0, 0), (0, pad))).reshape(_NW, NRV, 128)
    tg_v = jnp.pad(tg_w, ((0, 0), (0, pad))).reshape(_NW, NRV, 128)

    lse = pl.pallas_call(
        _lse_body,
        out_shape=jax.ShapeDtypeStruct((V, 1), jnp.float32),
    )(table)
    lse_pad = jnp.pad(lse.reshape(V), (0, 1024 - V))

    staging, parts = _make_sc_main(V, BT, 64)(tsw, in_g, in_v, tg_v, lse_pad)
    logits = _make_relayout(V, B, T, 16)(staging)
    loss = jnp.sum(parts / BT)
    return logits, loss


# speedup vs baseline: 2.8268x; 1.0211x over previous
import functools

import jax
import jax.numpy as jnp
from jax import lax
from jax.experimental import pallas as pl
from jax.experimental.pallas import tpu as pltpu, tpu_sc as plsc

_NC = 2
_NS = 16
_L = 16
_NW = _NC * _NS


def _lse_body(x_ref, lse_ref):
    x = x_ref[...]
    m = jnp.max(x, axis=1, keepdims=True)
    s = jnp.sum(jnp.exp(x - m), axis=1, keepdims=True)
    lse_ref[...] = m + jnp.log(s)


def _make_sc_main(V, BT, CH):
    PW = BT // _NW
    NCH = PW // CH
    NRV = (PW + 127) // 128
    mesh = plsc.VectorSubcoreMesh(core_axis_name="c", subcore_axis_name="s")

    NV = (CH + _L - 1) // _L

    @functools.partial(
        pl.kernel,
        out_type=(
            jax.ShapeDtypeStruct((BT, 8, 128), jnp.float32),
            jax.ShapeDtypeStruct((_NW, _L), jnp.float32),
        ),
        mesh=mesh,
        compiler_params=pltpu.CompilerParams(needs_layout_passes=False),
        scratch_types=[
            pltpu.VMEM((NCH, CH), jnp.int32),
            pltpu.VMEM((NRV, 128), jnp.int32),
            pltpu.VMEM((NRV, 128), jnp.int32),
            pltpu.VMEM((1024,), jnp.float32),
            pltpu.VMEM((CH, 8, 128), jnp.float32),
            pltpu.VMEM((CH, 8, 128), jnp.float32),
            pltpu.VMEM((_L,), jnp.float32),
            pltpu.SemaphoreType.DMA,
            pltpu.SemaphoreType.DMA,
            pltpu.SemaphoreType.DMA,
            pltpu.SemaphoreType.DMA,
        ],
    )
    def sc_main(tsw_hbm, ing_hbm, inv_hbm, tgt_hbm, lse_hbm,
                out_hbm, part_hbm,
                idxg_v, idx_v, tgt_v, lse_v, rows_a, rows_b, part_v,
                gsa, gsb, wsa, wsb):
        wid = lax.axis_index("s") * _NC + lax.axis_index("c")
        base = wid * PW
        pltpu.sync_copy(ing_hbm.at[wid], idxg_v)
        pltpu.sync_copy(inv_hbm.at[wid], idx_v)
        pltpu.sync_copy(tgt_hbm.at[wid], tgt_v)
        pltpu.sync_copy(lse_hbm, lse_v)

        def lse_step(s, acc):
            p = s * _L + lax.iota(jnp.int32, _L)
            iv = plsc.load_gather(idx_v, [p >> 7, p & 127])
            return acc + plsc.load_gather(lse_v, [iv])

        acc = lax.fori_loop(0, PW // _L, lse_step, jnp.zeros((_L,), jnp.float32))

        def picks(g, rows_v, acc):
            for h in range(NV):
                jvec = h * _L + lax.iota(jnp.int32, _L)
                m = jvec < CH
                jc = jnp.minimum(jvec, CH - 1)
                p = jnp.minimum(g * CH + jvec, PW - 1)
                tv = plsc.load_gather(tgt_v, [p >> 7, p & 127])
                pick = plsc.load_gather(rows_v, [jc, tv >> 7, tv & 127], mask=m)
                acc = acc - jnp.where(m, pick, jnp.zeros((_L,), jnp.float32))
            return acc

        def gather(g, rows_v, sem):
            pltpu.async_copy(tsw_hbm.at[idxg_v.at[g]], rows_v, sem)

        def gwait(rows_v, sem):
            pltpu.make_async_copy(tsw_hbm.at[idxg_v.at[0]], rows_v, sem).wait()

        def wstart(g, rows_v, sem):
            pltpu.async_copy(rows_v, out_hbm.at[pl.ds(base + g * CH, CH)], sem)

        def wwait(rows_v, sem):
            pltpu.make_async_copy(rows_v, out_hbm.at[pl.ds(base, CH)], sem).wait()

        gather(0, rows_a, gsa)

        def pair_step(i, acc):
            a = 2 * i
            gather(a + 1, rows_b, gsb)
            gwait(rows_a, gsa)
            acc = picks(a, rows_a, acc)
            wstart(a, rows_a, wsa)

            @pl.when(i < NCH // 2 - 1)
            def _():
                wwait(rows_a, wsa)
                gather(a + 2, rows_a, gsa)

            gwait(rows_b, gsb)
            acc = picks(a + 1, rows_b, acc)
            wstart(a + 1, rows_b, wsb)

            @pl.when(i < NCH // 2 - 1)
            def _():
                wwait(rows_b, wsb)

            return acc

        acc = lax.fori_loop(0, NCH // 2, pair_step, acc)
        wwait(rows_a, wsa)
        wwait(rows_b, wsb)

        part_v[...] = acc
        pltpu.sync_copy(part_v, part_hbm.at[wid])

    return sc_main


def _make_relayout(V, B, T, NB):
    def body(x_ref, o_ref):
        x = x_ref[...].reshape(NB * T, 1024)
        o_ref[...] = x[:, :V].reshape(NB, T, V)

    return pl.pallas_call(
        body,
        grid=(B // NB,),
        in_specs=[pl.BlockSpec((NB * T, 8, 128), lambda i: (i, 0, 0))],
        out_specs=pl.BlockSpec((NB, T, V), lambda i: (i, 0, 0)),
        out_shape=jax.ShapeDtypeStruct((B, T, V), jnp.float32),
        compiler_params=pltpu.CompilerParams(
            dimension_semantics=("arbitrary",),
        ),
    )


def kernel(table, inputs, targets):
    V = table.shape[0]
    B, T = inputs.shape
    BT = B * T
    PW = BT // _NW
    NRV = (PW + 127) // 128
    tsw = jnp.pad(table, ((0, 0), (0, 1024 - V))).reshape(V, 8, 128)

    in_w = inputs.astype(jnp.int32).reshape(_NW, PW)
    tg_w = targets.astype(jnp.int32).reshape(_NW, PW)
    in_g = in_w.reshape(_NW, PW // 50, 50)
    pad = NRV * 128 - PW
    in_v = jnp.pad(in_w, ((0, 0), (0, pad))).reshape(_NW, NRV, 128)
    tg_v = jnp.pad(tg_w, ((0, 0), (0, pad))).reshape(_NW, NRV, 128)

    lse = pl.pallas_call(
        _lse_body,
        out_shape=jax.ShapeDtypeStruct((V, 1), jnp.float32),
    )(table)
    lse_pad = jnp.pad(lse.reshape(V), (0, 1024 - V))

    staging, parts = _make_sc_main(V, BT, 50)(tsw, in_g, in_v, tg_v, lse_pad)
    logits = staging.reshape(B, T, 1024)[:, :, :V]
    loss = jnp.sum(parts / BT)
    return logits, loss


# speedup vs baseline: 3.0895x; 1.0929x over previous
import functools

import jax
import jax.numpy as jnp
from jax import lax
from jax.experimental import pallas as pl
from jax.experimental.pallas import tpu as pltpu, tpu_sc as plsc

_NC = 2
_NS = 16
_L = 16
_NW = _NC * _NS


def _lse_body(x_ref, lse_ref):
    x = x_ref[...]
    m = jnp.max(x, axis=1, keepdims=True)
    s = jnp.sum(jnp.exp(x - m), axis=1, keepdims=True)
    lse_ref[...] = m + jnp.log(s)


def _make_sc_main(V, BT, CH):
    PW = BT // _NW
    NCH = PW // CH
    NRV = (PW + 127) // 128
    mesh = plsc.VectorSubcoreMesh(core_axis_name="c", subcore_axis_name="s")

    NV = (CH + _L - 1) // _L

    @functools.partial(
        pl.kernel,
        out_type=(
            jax.ShapeDtypeStruct((8, BT, 128), jnp.float32),
            jax.ShapeDtypeStruct((_NW, _L), jnp.float32),
        ),
        mesh=mesh,
        compiler_params=pltpu.CompilerParams(needs_layout_passes=False),
        scratch_types=[
            pltpu.VMEM((NCH * 8, CH), jnp.int32),
            pltpu.VMEM((NRV, 128), jnp.int32),
            pltpu.VMEM((NRV, 128), jnp.int32),
            pltpu.VMEM((1024,), jnp.float32),
            pltpu.VMEM((8, CH, 128), jnp.float32),
            pltpu.VMEM((8, CH, 128), jnp.float32),
            pltpu.VMEM((_L,), jnp.float32),
            pltpu.SemaphoreType.DMA,
            pltpu.SemaphoreType.DMA,
            pltpu.SemaphoreType.DMA,
            pltpu.SemaphoreType.DMA,
        ],
    )
    def sc_main(tsw_hbm, ing_hbm, inv_hbm, tgt_hbm, lse_hbm,
                out_hbm, part_hbm,
                idxg_v, idx_v, tgt_v, lse_v, rows_a, rows_b, part_v,
                gsa, gsb, wsa, wsb):
        wid = lax.axis_index("s") * _NC + lax.axis_index("c")
        base = wid * PW
        pltpu.sync_copy(ing_hbm.at[wid], idxg_v)
        pltpu.sync_copy(inv_hbm.at[wid], idx_v)
        pltpu.sync_copy(tgt_hbm.at[wid], tgt_v)
        pltpu.sync_copy(lse_hbm, lse_v)

        def lse_step(s, acc):
            p = s * _L + lax.iota(jnp.int32, _L)
            iv = plsc.load_gather(idx_v, [p >> 7, p & 127])
            return acc + plsc.load_gather(lse_v, [iv])

        acc = lax.fori_loop(0, PW // _L, lse_step, jnp.zeros((_L,), jnp.float32))

        def picks(g, rows_v, acc):
            for h in range(NV):
                jvec = h * _L + lax.iota(jnp.int32, _L)
                m = jvec < CH
                jc = jnp.minimum(jvec, CH - 1)
                p = jnp.minimum(g * CH + jvec, PW - 1)
                tv = plsc.load_gather(tgt_v, [p >> 7, p & 127])
                pick = plsc.load_gather(rows_v, [tv >> 7, jc, tv & 127], mask=m)
                acc = acc - jnp.where(m, pick, jnp.zeros((_L,), jnp.float32))
            return acc

        def gather(g, rows_v, sem):
            for sp in range(8):
                pltpu.async_copy(
                    tsw_hbm.at[idxg_v.at[g * 8 + sp]], rows_v.at[sp], sem)

        def gwait(rows_v, sem):
            for sp in range(8):
                pltpu.make_async_copy(
                    tsw_hbm.at[idxg_v.at[sp]], rows_v.at[sp], sem).wait()

        def wstart(g, rows_v, sem):
            pltpu.async_copy(
                rows_v, out_hbm.at[:, pl.ds(base + g * CH, CH), :], sem)

        def wwait(rows_v, sem):
            pltpu.make_async_copy(
                rows_v, out_hbm.at[:, pl.ds(base, CH), :], sem).wait()

        gather(0, rows_a, gsa)

        def pair_step(i, acc):
            a = 2 * i
            gather(a + 1, rows_b, gsb)
            gwait(rows_a, gsa)
            acc = picks(a, rows_a, acc)
            wstart(a, rows_a, wsa)

            @pl.when(i < NCH // 2 - 1)
            def _():
                wwait(rows_a, wsa)
                gather(a + 2, rows_a, gsa)

            gwait(rows_b, gsb)
            acc = picks(a + 1, rows_b, acc)
            wstart(a + 1, rows_b, wsb)

            @pl.when(i < NCH // 2 - 1)
            def _():
                wwait(rows_b, wsb)

            return acc

        acc = lax.fori_loop(0, NCH // 2, pair_step, acc)
        wwait(rows_a, wsa)
        wwait(rows_b, wsb)

        part_v[...] = acc
        pltpu.sync_copy(part_v, part_hbm.at[wid])

    return sc_main


def _make_relayout(V, B, T, NB):
    def body(x_ref, o_ref):
        x = x_ref[...].reshape(NB * T, 1024)
        o_ref[...] = x[:, :V].reshape(NB, T, V)

    return pl.pallas_call(
        body,
        grid=(B // NB,),
        in_specs=[pl.BlockSpec((NB * T, 8, 128), lambda i: (i, 0, 0))],
        out_specs=pl.BlockSpec((NB, T, V), lambda i: (i, 0, 0)),
        out_shape=jax.ShapeDtypeStruct((B, T, V), jnp.float32),
        compiler_params=pltpu.CompilerParams(
            dimension_semantics=("arbitrary",),
        ),
    )


def _make_transpose(V, B, T):
    def body(x_ref, o_ref):
        x = x_ref[...].reshape(128, T, 128)
        o_ref[...] = jnp.transpose(x, (1, 2, 0))

    return pl.pallas_call(
        body,
        grid=(B // 128, 8),
        in_specs=[pl.BlockSpec((1, 128 * T, 128), lambda i, j: (j, i, 0))],
        out_specs=pl.BlockSpec((T, 128, 128), lambda i, j: (0, j, i)),
        out_shape=jax.ShapeDtypeStruct((T, V, B), jnp.float32),
        compiler_params=pltpu.CompilerParams(
            dimension_semantics=("arbitrary", "arbitrary"),
        ),
    )


def kernel(table, inputs, targets):
    V = table.shape[0]
    B, T = inputs.shape
    BT = B * T
    PW = BT // _NW
    NRV = (PW + 127) // 128
    tsw = jnp.pad(table, ((0, 0), (0, 1024 - V)))
    tsw2 = tsw.reshape(V, 8, 128).transpose(1, 0, 2).reshape(8 * V, 128)

    CH = 40
    in_w = inputs.astype(jnp.int32).reshape(_NW, PW)
    tg_w = targets.astype(jnp.int32).reshape(_NW, PW)
    in_g = (in_w.reshape(_NW, PW // CH, 1, CH)
            + (jnp.arange(8, dtype=jnp.int32) * V)[None, None, :, None]
            ).reshape(_NW, (PW // CH) * 8, CH)
    pad = NRV * 128 - PW
    in_v = jnp.pad(in_w, ((0, 0), (0, pad))).reshape(_NW, NRV, 128)
    tg_v = jnp.pad(tg_w, ((0, 0), (0, pad))).reshape(_NW, NRV, 128)

    lse = pl.pallas_call(
        _lse_body,
        out_shape=jax.ShapeDtypeStruct((V, 1), jnp.float32),
    )(table)
    lse_pad = jnp.pad(lse.reshape(V), (0, 1024 - V))

    staging, parts = _make_sc_main(V, BT, CH)(tsw2, in_g, in_v, tg_v, lse_pad)
    logits = jnp.transpose(_make_transpose(V, B, T)(staging), (2, 0, 1))
    loss = jnp.sum(parts / BT)
    return logits, loss
